# baseline (speedup 1.0000x reference)
import functools

import jax
import jax.numpy as jnp
from jax import lax
from jax.experimental import pallas as pl
from jax.experimental.pallas import tpu as pltpu
from jax.experimental.pallas import tpu_sc as plsc

N = 10000
E = 320000
D = 128
EPS = 1e-5

NC = 2
NS = 16
NW = NC * NS
C = 128

CHUNKS_PER_W = 80
BCH = 40
EPW = CHUNKS_PER_W * C
EP = EPW * NW
NP = 10112
INIT_ROWS = NP // NS
OUT_ROWS = 632
OUT_TAIL = N - 15 * OUT_ROWS


def _sc_segment_sum(src_p, dst_p, feats, zeros):
    mesh = plsc.VectorSubcoreMesh(core_axis_name="c", subcore_axis_name="s")

    @functools.partial(
        pl.kernel,
        out_type=jax.ShapeDtypeStruct((NC, N, D), jnp.float32),
        mesh=mesh,
        scratch_types=[
            pltpu.VMEM((C,), jnp.int32),
            pltpu.VMEM((C,), jnp.int32),
            pltpu.VMEM((C, D), jnp.float32),
            pltpu.VMEM((C, D), jnp.float32),
            pltpu.VMEM_SHARED((NP, D), jnp.float32),
            pltpu.SemaphoreType.DMA,
            pltpu.SemaphoreType.DMA,
        ],
    )
    def seg_sum(src_hbm, dst_hbm, feats_hbm, zeros_hbm, out_hbm,
                src_v, dst_v, rows0, rows1, acc_sh, sem0, sem1):
        cid = lax.axis_index("c")
        sid = lax.axis_index("s")
        wid = sid * NC + cid
        pltpu.sync_copy(zeros_hbm.at[pl.ds(sid * INIT_ROWS, INIT_ROWS)],
                        acc_sh.at[pl.ds(sid * INIT_ROWS, INIT_ROWS)])
        plsc.subcore_barrier()

        base = wid * EPW

        @pl.loop(0, CHUNKS_PER_W)
        def _(j):
            off = base + j * C
            pltpu.sync_copy(src_hbm.at[pl.ds(off, C)], src_v)
            pltpu.sync_copy(dst_hbm.at[pl.ds(off, C)], dst_v)
            pltpu.async_copy(feats_hbm.at[src_v], rows0, sem0).wait()
            pltpu.sync_copy(rows0, acc_sh.at[dst_v], add=True)

        plsc.subcore_barrier()

        @pl.when(sid < NS - 1)
        def _():
            pltpu.sync_copy(acc_sh.at[pl.ds(sid * OUT_ROWS, OUT_ROWS)],
                            out_hbm.at[cid, pl.ds(sid * OUT_ROWS, OUT_ROWS)])

        @pl.when(sid == NS - 1)
        def _():
            pltpu.sync_copy(acc_sh.at[pl.ds((NS - 1) * OUT_ROWS, OUT_TAIL)],
                            out_hbm.at[cid, pl.ds((NS - 1) * OUT_ROWS, OUT_TAIL)])

    return seg_sum(src_p, dst_p, feats, zeros)


R = 1000
NBLK = N // R


def _tc_fused_body(p0_ref, p1_ref, f_ref, w_ref, b_ref, wr_ref, br_ref,
                   h_ref, stats_ref, acc_ref):
    i = pl.program_id(0)
    agg = p0_ref[...] + p1_ref[...]
    h = jnp.dot(agg, w_ref[...], preferred_element_type=jnp.float32)
    h = jnp.maximum(h + b_ref[...], 0.0)
    r = jnp.dot(f_ref[...], wr_ref[...], preferred_element_type=jnp.float32)
    r = jnp.maximum(r + br_ref[...], 0.0)
    h = h + r
    h_ref[...] = h

    @pl.when(i == 0)
    def _():
        acc_ref[...] = jnp.zeros_like(acc_ref)

    acc_ref[0:1, :] += jnp.sum(h, axis=0, keepdims=True)
    acc_ref[1:2, :] += jnp.sum(h * h, axis=0, keepdims=True)

    @pl.when(i == NBLK - 1)
    def _():
        stats_ref[...] = acc_ref[...]


def _tc_norm_body(h_ref, stats_ref, g_ref, bt_ref, o_ref):
    mean = stats_ref[0:1, :] * (1.0 / N)
    var = stats_ref[1:2, :] * (1.0 / N) - mean * mean
    inv = lax.rsqrt(var + EPS)
    o_ref[...] = (h_ref[...] - mean) * (inv * g_ref[...]) + bt_ref[...]


def kernel(feats, edge_index, W, b, W_res, b_res, gamma, beta):
    src = edge_index[0].astype(jnp.int32)
    dst = edge_index[1].astype(jnp.int32)
    pad = EP - E
    src_p = jnp.concatenate([src, jnp.zeros((pad,), jnp.int32)])
    dst_p = jnp.concatenate([dst, jnp.full((pad,), N, jnp.int32)])
    zeros = jnp.zeros((NP, D), jnp.float32)

    parts = _sc_segment_sum(src_p, dst_p, feats, zeros)
    p0, p1 = parts[0], parts[1]

    blk = lambda i: (i, 0)
    full = lambda i: (0, 0)
    h_pre, stats = pl.pallas_call(
        _tc_fused_body,
        grid=(NBLK,),
        in_specs=[
            pl.BlockSpec((R, D), blk),
            pl.BlockSpec((R, D), blk),
            pl.BlockSpec((R, D), blk),
            pl.BlockSpec((D, D), full),
            pl.BlockSpec((1, D), full),
            pl.BlockSpec((D, D), full),
            pl.BlockSpec((1, D), full),
        ],
        out_specs=[
            pl.BlockSpec((R, D), blk),
            pl.BlockSpec((2, D), full),
        ],
        out_shape=[
            jax.ShapeDtypeStruct((N, D), jnp.float32),
            jax.ShapeDtypeStruct((2, D), jnp.float32),
        ],
        scratch_shapes=[pltpu.VMEM((2, D), jnp.float32)],
    )(p0, p1, feats, W, b.reshape(1, D), W_res, b_res.reshape(1, D))

    out = pl.pallas_call(
        _tc_norm_body,
        grid=(NBLK,),
        in_specs=[
            pl.BlockSpec((R, D), blk),
            pl.BlockSpec((2, D), full),
            pl.BlockSpec((1, D), full),
            pl.BlockSpec((1, D), full),
        ],
        out_specs=pl.BlockSpec((R, D), blk),
        out_shape=jax.ShapeDtypeStruct((N, D), jnp.float32),
    )(h_pre, stats, gamma.reshape(1, D), beta.reshape(1, D))
    return out


# speedup vs baseline: 1.5100x; 1.5100x over previous
import functools

import numpy as np

import jax
import jax.numpy as jnp
from jax import lax
from jax.experimental import pallas as pl
from jax.experimental.pallas import tpu as pltpu
from jax.experimental.pallas import tpu_sc as plsc

N = 10000
E = 320000
D = 128
EPS = 1e-5

NC = 2
NS = 16
NW = NC * NS
C = 128

CHUNKS_PER_W = 80
BCH = 40
EPW = CHUNKS_PER_W * C
EP = EPW * NW
NP = 10112
INIT_ROWS = NP // NS
OUT_ROWS = 632
OUT_TAIL = N - 15 * OUT_ROWS


def _sc_segment_sum(src_p, dst_p, feats_i32, zeros):
    mesh = plsc.VectorSubcoreMesh(core_axis_name="c", subcore_axis_name="s")
    D2 = D // 2

    @functools.partial(
        pl.kernel,
        out_type=jax.ShapeDtypeStruct((NC, N, D), jnp.float32),
        mesh=mesh,
        compiler_params=pltpu.CompilerParams(use_tc_tiling_on_sc=False),
        scratch_types=[
            pltpu.VMEM((C,), jnp.int32),
            pltpu.VMEM((C,), jnp.int32),
            pltpu.VMEM((C,), jnp.int32),
            pltpu.VMEM((C, D2), jnp.int32),
            pltpu.VMEM((C, D2), jnp.int32),
            pltpu.VMEM((C, D), jnp.float32),
            pltpu.VMEM_SHARED((NP, D), jnp.float32),
            pltpu.SemaphoreType.DMA,
            pltpu.SemaphoreType.DMA,
        ],
    )
    def seg_sum(src_hbm, dst_hbm, feats_hbm, zeros_hbm, out_hbm,
                src0_v, src1_v, dst_v, bf0, bf1, rows_f, acc_sh, sem0, sem1):
        cid = lax.axis_index("c")
        sid = lax.axis_index("s")
        wid = sid * NC + cid
        pltpu.sync_copy(zeros_hbm.at[pl.ds(sid * INIT_ROWS, INIT_ROWS)],
                        acc_sh.at[pl.ds(sid * INIT_ROWS, INIT_ROWS)])
        plsc.subcore_barrier()

        base = wid * EPW

        def unpack_rows(bf_ref):
            @pl.loop(0, C)
            def _(r):
                for g in range(D2 // 16):
                    v = bf_ref[r, pl.ds(16 * g, 16)]
                    a = lax.bitcast_convert_type(v << 16, jnp.float32)
                    b2 = lax.bitcast_convert_type(
                        v & jnp.int32(-65536), jnp.float32)
                    rows_f[r, pl.ds(32 * g, 16)] = a
                    rows_f[r, pl.ds(32 * g + 16, 16)] = b2

        pltpu.sync_copy(src_hbm.at[pl.ds(base, C)], src0_v)
        pltpu.async_copy(feats_hbm.at[src0_v], bf0, sem0)
        NPAIR = CHUNKS_PER_W // 2

        @pl.loop(0, NPAIR)
        def _(p):
            off0 = base + 2 * p * C
            pltpu.sync_copy(src_hbm.at[pl.ds(off0 + C, C)], src1_v)
            pltpu.async_copy(feats_hbm.at[src1_v], bf1, sem1)
            pltpu.sync_copy(dst_hbm.at[pl.ds(off0, C)], dst_v)
            pltpu.make_async_copy(feats_hbm.at[src0_v], bf0, sem0).wait()
            unpack_rows(bf0)
            pltpu.sync_copy(rows_f, acc_sh.at[dst_v], add=True)

            @pl.when(p < NPAIR - 1)
            def _():
                pltpu.sync_copy(src_hbm.at[pl.ds(off0 + 2 * C, C)], src0_v)
                pltpu.async_copy(feats_hbm.at[src0_v], bf0, sem0)

            pltpu.sync_copy(dst_hbm.at[pl.ds(off0 + C, C)], dst_v)
            pltpu.make_async_copy(feats_hbm.at[src1_v], bf1, sem1).wait()
            unpack_rows(bf1)
            pltpu.sync_copy(rows_f, acc_sh.at[dst_v], add=True)

        plsc.subcore_barrier()

        @pl.when(sid < NS - 1)
        def _():
            pltpu.sync_copy(acc_sh.at[pl.ds(sid * OUT_ROWS, OUT_ROWS)],
                            out_hbm.at[cid, pl.ds(sid * OUT_ROWS, OUT_ROWS)])

        @pl.when(sid == NS - 1)
        def _():
            pltpu.sync_copy(acc_sh.at[pl.ds((NS - 1) * OUT_ROWS, OUT_TAIL)],
                            out_hbm.at[cid, pl.ds((NS - 1) * OUT_ROWS, OUT_TAIL)])

    return seg_sum(src_p, dst_p, feats_i32, zeros)


R = 1000
NBLK = N // R


def _tc_fused_body(p0_ref, p1_ref, f_ref, w_ref, b_ref, wr_ref, br_ref,
                   h_ref, stats_ref, acc_ref):
    i = pl.program_id(0)
    agg = p0_ref[...] + p1_ref[...]
    h = jnp.dot(agg, w_ref[...], preferred_element_type=jnp.float32)
    h = jnp.maximum(h + b_ref[...], 0.0)
    r = jnp.dot(f_ref[...], wr_ref[...], preferred_element_type=jnp.float32)
    r = jnp.maximum(r + br_ref[...], 0.0)
    h = h + r
    h_ref[...] = h

    @pl.when(i == 0)
    def _():
        acc_ref[...] = jnp.zeros_like(acc_ref)

    acc_ref[0:1, :] += jnp.sum(h, axis=0, keepdims=True)
    acc_ref[1:2, :] += jnp.sum(h * h, axis=0, keepdims=True)

    @pl.when(i == NBLK - 1)
    def _():
        stats_ref[...] = acc_ref[...]


def _tc_norm_body(h_ref, stats_ref, g_ref, bt_ref, o_ref):
    mean = stats_ref[0:1, :] * (1.0 / N)
    var = stats_ref[1:2, :] * (1.0 / N) - mean * mean
    inv = lax.rsqrt(var + EPS)
    o_ref[...] = (h_ref[...] - mean) * (inv * g_ref[...]) + bt_ref[...]


def kernel(feats, edge_index, W, b, W_res, b_res, gamma, beta):
    src = edge_index[0].astype(jnp.int32)
    dst = edge_index[1].astype(jnp.int32)
    pad = EP - E
    src_p = jnp.concatenate([src, jnp.zeros((pad,), jnp.int32)])
    dst_p = jnp.concatenate([dst, jnp.full((pad,), N, jnp.int32)])
    zeros = jnp.zeros((NP, D), jnp.float32)

    srccols = np.arange(D).reshape(D // 32, 2, 16).transpose(0, 2, 1)
    srccols = jnp.asarray(srccols.reshape(D), dtype=jnp.int32)
    feats_bf = feats[:, srccols].astype(jnp.bfloat16)
    feats_i32 = lax.bitcast_convert_type(
        feats_bf.reshape(N, D // 2, 2), jnp.int32)

    parts = _sc_segment_sum(src_p, dst_p, feats_i32, zeros)
    p0, p1 = parts[0], parts[1]

    blk = lambda i: (i, 0)
    full = lambda i: (0, 0)
    h_pre, stats = pl.pallas_call(
        _tc_fused_body,
        grid=(NBLK,),
        in_specs=[
            pl.BlockSpec((R, D), blk),
            pl.BlockSpec((R, D), blk),
            pl.BlockSpec((R, D), blk),
            pl.BlockSpec((D, D), full),
            pl.BlockSpec((1, D), full),
            pl.BlockSpec((D, D), full),
            pl.BlockSpec((1, D), full),
        ],
        out_specs=[
            pl.BlockSpec((R, D), blk),
            pl.BlockSpec((2, D), full),
        ],
        out_shape=[
            jax.ShapeDtypeStruct((N, D), jnp.float32),
            jax.ShapeDtypeStruct((2, D), jnp.float32),
        ],
        scratch_shapes=[pltpu.VMEM((2, D), jnp.float32)],
    )(p0, p1, feats, W, b.reshape(1, D), W_res, b_res.reshape(1, D))

    out = pl.pallas_call(
        _tc_norm_body,
        grid=(NBLK,),
        in_specs=[
            pl.BlockSpec((R, D), blk),
            pl.BlockSpec((2, D), full),
            pl.BlockSpec((1, D), full),
            pl.BlockSpec((1, D), full),
        ],
        out_specs=pl.BlockSpec((R, D), blk),
        out_shape=jax.ShapeDtypeStruct((N, D), jnp.float32),
    )(h_pre, stats, gamma.reshape(1, D), beta.reshape(1, D))
    return out


# speedup vs baseline: 1.6417x; 1.0872x over previous
import functools

import numpy as np

import jax
import jax.numpy as jnp
from jax import lax
from jax.experimental import pallas as pl
from jax.experimental.pallas import tpu as pltpu
from jax.experimental.pallas import tpu_sc as plsc

N = 10000
E = 320000
D = 128
EPS = 1e-5

NC = 2
NS = 16
NW = NC * NS
C = 128

CHUNKS_PER_W = 80
BCH = 40
EPW = CHUNKS_PER_W * C
EP = EPW * NW
NP = 10112
INIT_ROWS = NP // NS
OUT_ROWS = 632
OUT_TAIL = N - 15 * OUT_ROWS


def _sc_segment_sum(src_p, dst_p, feats_i32, zeros):
    mesh = plsc.VectorSubcoreMesh(core_axis_name="c", subcore_axis_name="s")
    D2 = D // 2

    @functools.partial(
        pl.kernel,
        out_type=jax.ShapeDtypeStruct((NC, N, D), jnp.float32),
        mesh=mesh,
        compiler_params=pltpu.CompilerParams(use_tc_tiling_on_sc=False),
        scratch_types=[
            pltpu.VMEM((C,), jnp.int32),
            pltpu.VMEM((C,), jnp.int32),
            pltpu.VMEM((C,), jnp.int32),
            pltpu.VMEM((C,), jnp.int32),
            pltpu.VMEM((C, D2), jnp.int32),
            pltpu.VMEM((C, D2), jnp.int32),
            pltpu.VMEM((C, D), jnp.float32),
            pltpu.VMEM((C, D), jnp.float32),
            pltpu.VMEM_SHARED((NP, D), jnp.float32),
            pltpu.SemaphoreType.DMA,
            pltpu.SemaphoreType.DMA,
            pltpu.SemaphoreType.DMA,
            pltpu.SemaphoreType.DMA,
        ],
    )
    def seg_sum(src_hbm, dst_hbm, feats_hbm, zeros_hbm, out_hbm,
                src0_v, src1_v, dst0_v, dst1_v, bf0, bf1, rf0, rf1,
                acc_sh, sem0, sem1, sem2, sem3):
        cid = lax.axis_index("c")
        sid = lax.axis_index("s")
        wid = sid * NC + cid
        pltpu.sync_copy(zeros_hbm.at[pl.ds(sid * INIT_ROWS, INIT_ROWS)],
                        acc_sh.at[pl.ds(sid * INIT_ROWS, INIT_ROWS)])
        plsc.subcore_barrier()

        base = wid * EPW

        def unpack_rows(bf_ref, rf_ref):
            @pl.loop(0, C, unroll=4)
            def _(r):
                for g in range(D2 // 16):
                    v = bf_ref[r, pl.ds(16 * g, 16)]
                    a = lax.bitcast_convert_type(v << 16, jnp.float32)
                    b2 = lax.bitcast_convert_type(
                        v & jnp.int32(-65536), jnp.float32)
                    rf_ref[r, pl.ds(32 * g, 16)] = a
                    rf_ref[r, pl.ds(32 * g + 16, 16)] = b2

        pltpu.sync_copy(src_hbm.at[pl.ds(base, C)], src0_v)
        pltpu.async_copy(feats_hbm.at[src0_v], bf0, sem0)
        NPAIR = CHUNKS_PER_W // 2

        @pl.loop(0, NPAIR)
        def _(p):
            off0 = base + 2 * p * C
            pltpu.sync_copy(src_hbm.at[pl.ds(off0 + C, C)], src1_v)
            pltpu.async_copy(feats_hbm.at[src1_v], bf1, sem1)
            pltpu.make_async_copy(feats_hbm.at[src0_v], bf0, sem0).wait()

            @pl.when(p > 0)
            def _():
                pltpu.make_async_copy(rf0, acc_sh.at[dst0_v], sem2).wait()

            unpack_rows(bf0, rf0)
            pltpu.sync_copy(dst_hbm.at[pl.ds(off0, C)], dst0_v)
            pltpu.make_async_copy(rf0, acc_sh.at[dst0_v], sem2).start(
                add=True)

            @pl.when(p < NPAIR - 1)
            def _():
                pltpu.sync_copy(src_hbm.at[pl.ds(off0 + 2 * C, C)], src0_v)
                pltpu.async_copy(feats_hbm.at[src0_v], bf0, sem0)

            pltpu.make_async_copy(feats_hbm.at[src1_v], bf1, sem1).wait()

            @pl.when(p > 0)
            def _():
                pltpu.make_async_copy(rf1, acc_sh.at[dst1_v], sem3).wait()

            unpack_rows(bf1, rf1)
            pltpu.sync_copy(dst_hbm.at[pl.ds(off0 + C, C)], dst1_v)
            pltpu.make_async_copy(rf1, acc_sh.at[dst1_v], sem3).start(
                add=True)

        pltpu.make_async_copy(rf0, acc_sh.at[dst0_v], sem2).wait()
        pltpu.make_async_copy(rf1, acc_sh.at[dst1_v], sem3).wait()
        plsc.subcore_barrier()

        @pl.when(sid < NS - 1)
        def _():
            pltpu.sync_copy(acc_sh.at[pl.ds(sid * OUT_ROWS, OUT_ROWS)],
                            out_hbm.at[cid, pl.ds(sid * OUT_ROWS, OUT_ROWS)])

        @pl.when(sid == NS - 1)
        def _():
            pltpu.sync_copy(acc_sh.at[pl.ds((NS - 1) * OUT_ROWS, OUT_TAIL)],
                            out_hbm.at[cid, pl.ds((NS - 1) * OUT_ROWS, OUT_TAIL)])

    return seg_sum(src_p, dst_p, feats_i32, zeros)


R = 1000
NBLK = N // R


def _tc_fused_body(p0_ref, p1_ref, f_ref, w_ref, b_ref, wr_ref, br_ref,
                   h_ref, stats_ref, acc_ref):
    i = pl.program_id(0)
    agg = p0_ref[...] + p1_ref[...]
    h = jnp.dot(agg, w_ref[...], preferred_element_type=jnp.float32)
    h = jnp.maximum(h + b_ref[...], 0.0)
    r = jnp.dot(f_ref[...], wr_ref[...], preferred_element_type=jnp.float32)
    r = jnp.maximum(r + br_ref[...], 0.0)
    h = h + r
    h_ref[...] = h

    @pl.when(i == 0)
    def _():
        acc_ref[...] = jnp.zeros_like(acc_ref)

    acc_ref[0:1, :] += jnp.sum(h, axis=0, keepdims=True)
    acc_ref[1:2, :] += jnp.sum(h * h, axis=0, keepdims=True)

    @pl.when(i == NBLK - 1)
    def _():
        stats_ref[...] = acc_ref[...]


def _tc_norm_body(h_ref, stats_ref, g_ref, bt_ref, o_ref):
    mean = stats_ref[0:1, :] * (1.0 / N)
    var = stats_ref[1:2, :] * (1.0 / N) - mean * mean
    inv = lax.rsqrt(var + EPS)
    o_ref[...] = (h_ref[...] - mean) * (inv * g_ref[...]) + bt_ref[...]


def kernel(feats, edge_index, W, b, W_res, b_res, gamma, beta):
    src = edge_index[0].astype(jnp.int32)
    dst = edge_index[1].astype(jnp.int32)
    pad = EP - E
    src_p = jnp.concatenate([src, jnp.zeros((pad,), jnp.int32)])
    dst_p = jnp.concatenate([dst, jnp.full((pad,), N, jnp.int32)])
    zeros = jnp.zeros((NP, D), jnp.float32)

    srccols = np.arange(D).reshape(D // 32, 2, 16).transpose(0, 2, 1)
    srccols = jnp.asarray(srccols.reshape(D), dtype=jnp.int32)
    feats_bf = feats[:, srccols].astype(jnp.bfloat16)
    feats_i32 = lax.bitcast_convert_type(
        feats_bf.reshape(N, D // 2, 2), jnp.int32)

    parts = _sc_segment_sum(src_p, dst_p, feats_i32, zeros)
    p0, p1 = parts[0], parts[1]

    blk = lambda i: (i, 0)
    full = lambda i: (0, 0)
    h_pre, stats = pl.pallas_call(
        _tc_fused_body,
        grid=(NBLK,),
        in_specs=[
            pl.BlockSpec((R, D), blk),
            pl.BlockSpec((R, D), blk),
            pl.BlockSpec((R, D), blk),
            pl.BlockSpec((D, D), full),
            pl.BlockSpec((1, D), full),
            pl.BlockSpec((D, D), full),
            pl.BlockSpec((1, D), full),
        ],
        out_specs=[
            pl.BlockSpec((R, D), blk),
            pl.BlockSpec((2, D), full),
        ],
        out_shape=[
            jax.ShapeDtypeStruct((N, D), jnp.float32),
            jax.ShapeDtypeStruct((2, D), jnp.float32),
        ],
        scratch_shapes=[pltpu.VMEM((2, D), jnp.float32)],
    )(p0, p1, feats, W, b.reshape(1, D), W_res, b_res.reshape(1, D))

    out = pl.pallas_call(
        _tc_norm_body,
        grid=(NBLK,),
        in_specs=[
            pl.BlockSpec((R, D), blk),
            pl.BlockSpec((2, D), full),
            pl.BlockSpec((1, D), full),
            pl.BlockSpec((1, D), full),
        ],
        out_specs=pl.BlockSpec((R, D), blk),
        out_shape=jax.ShapeDtypeStruct((N, D), jnp.float32),
    )(h_pre, stats, gamma.reshape(1, D), beta.reshape(1, D))
    return out


# speedup vs baseline: 1.7827x; 1.0859x over previous
import functools

import numpy as np

import jax
import jax.numpy as jnp
from jax import lax
from jax.experimental import pallas as pl
from jax.experimental.pallas import tpu as pltpu
from jax.experimental.pallas import tpu_sc as plsc

N = 10000
E = 320000
D = 128
EPS = 1e-5

NC = 2
NS = 16
NW = NC * NS
C = 128

CHUNKS_PER_W = 80
BCH = 40
EPW = CHUNKS_PER_W * C
EP = EPW * NW
NP = 10112
INIT_ROWS = NP // NS
OUT_ROWS = 632
OUT_TAIL = N - 15 * OUT_ROWS


def _sc_segment_sum(src_p, dst_p, feats_i32, zeros):
    mesh = plsc.VectorSubcoreMesh(core_axis_name="c", subcore_axis_name="s")
    D2 = D // 2

    @functools.partial(
        pl.kernel,
        out_type=jax.ShapeDtypeStruct((NC, N, D), jnp.float32),
        mesh=mesh,
        compiler_params=pltpu.CompilerParams(use_tc_tiling_on_sc=False),
        scratch_types=[
            pltpu.VMEM((C,), jnp.int32),
            pltpu.VMEM((C,), jnp.int32),
            pltpu.VMEM((C,), jnp.int32),
            pltpu.VMEM((C,), jnp.int32),
            pltpu.VMEM((C, D2), jnp.int32),
            pltpu.VMEM((C, D2), jnp.int32),
            pltpu.VMEM((C, D), jnp.float32),
            pltpu.VMEM((C, D), jnp.float32),
            pltpu.VMEM_SHARED((NP, D), jnp.float32),
            pltpu.SemaphoreType.DMA,
            pltpu.SemaphoreType.DMA,
            pltpu.SemaphoreType.DMA,
            pltpu.SemaphoreType.DMA,
        ],
    )
    def seg_sum(src_hbm, dst_hbm, feats_hbm, zeros_hbm, out_hbm,
                src0_v, src1_v, dst0_v, dst1_v, bf0, bf1, rf0, rf1,
                acc_sh, sem0, sem1, sem2, sem3):
        cid = lax.axis_index("c")
        sid = lax.axis_index("s")
        wid = sid * NC + cid
        pltpu.sync_copy(zeros_hbm,
                        acc_sh.at[pl.ds(sid * INIT_ROWS, INIT_ROWS)])
        plsc.subcore_barrier()

        base = wid * EPW

        def unpack_rows(bf_ref, rf_ref):
            @pl.loop(0, C, unroll=4)
            def _(r):
                for g in range(D2 // 16):
                    v = bf_ref[r, pl.ds(16 * g, 16)]
                    a = lax.bitcast_convert_type(v << 16, jnp.float32)
                    b2 = lax.bitcast_convert_type(
                        v & jnp.int32(-65536), jnp.float32)
                    rf_ref[r, pl.ds(32 * g, 16)] = a
                    rf_ref[r, pl.ds(32 * g + 16, 16)] = b2

        pltpu.sync_copy(src_hbm.at[pl.ds(base, C)], src0_v)
        pltpu.async_copy(feats_hbm.at[src0_v], bf0, sem0)
        NPAIR = CHUNKS_PER_W // 2

        @pl.loop(0, NPAIR)
        def _(p):
            off0 = base + 2 * p * C
            pltpu.sync_copy(src_hbm.at[pl.ds(off0 + C, C)], src1_v)
            pltpu.async_copy(feats_hbm.at[src1_v], bf1, sem1)
            pltpu.make_async_copy(feats_hbm.at[src0_v], bf0, sem0).wait()

            @pl.when(p > 0)
            def _():
                pltpu.make_async_copy(rf0, acc_sh.at[dst0_v], sem2).wait()

            unpack_rows(bf0, rf0)
            pltpu.sync_copy(dst_hbm.at[pl.ds(off0, C)], dst0_v)
            pltpu.make_async_copy(rf0, acc_sh.at[dst0_v], sem2).start(
                add=True)

            @pl.when(p < NPAIR - 1)
            def _():
                pltpu.sync_copy(src_hbm.at[pl.ds(off0 + 2 * C, C)], src0_v)
                pltpu.async_copy(feats_hbm.at[src0_v], bf0, sem0)

            pltpu.make_async_copy(feats_hbm.at[src1_v], bf1, sem1).wait()

            @pl.when(p > 0)
            def _():
                pltpu.make_async_copy(rf1, acc_sh.at[dst1_v], sem3).wait()

            unpack_rows(bf1, rf1)
            pltpu.sync_copy(dst_hbm.at[pl.ds(off0 + C, C)], dst1_v)
            pltpu.make_async_copy(rf1, acc_sh.at[dst1_v], sem3).start(
                add=True)

        pltpu.make_async_copy(rf0, acc_sh.at[dst0_v], sem2).wait()
        pltpu.make_async_copy(rf1, acc_sh.at[dst1_v], sem3).wait()
        plsc.subcore_barrier()

        @pl.when(sid < NS - 1)
        def _():
            pltpu.sync_copy(acc_sh.at[pl.ds(sid * OUT_ROWS, OUT_ROWS)],
                            out_hbm.at[cid, pl.ds(sid * OUT_ROWS, OUT_ROWS)])

        @pl.when(sid == NS - 1)
        def _():
            pltpu.sync_copy(acc_sh.at[pl.ds((NS - 1) * OUT_ROWS, OUT_TAIL)],
                            out_hbm.at[cid, pl.ds((NS - 1) * OUT_ROWS, OUT_TAIL)])

    return seg_sum(src_p, dst_p, feats_i32, zeros)


R = 1000
NBLK = N // R


def _tc_fused_body(p0_ref, p1_ref, f_ref, w_ref, b_ref, wr_ref, br_ref,
                   g_ref, bt_ref, o_ref, h_all, acc_ref):
    ph = pl.program_id(0)
    i = pl.program_id(1)

    @pl.when(ph == 0)
    def _():
        agg = p0_ref[...] + p1_ref[...]
        h = jnp.dot(agg, w_ref[...], preferred_element_type=jnp.float32)
        h = jnp.maximum(h + b_ref[...], 0.0)
        r = jnp.dot(f_ref[...], wr_ref[...],
                    preferred_element_type=jnp.float32)
        r = jnp.maximum(r + br_ref[...], 0.0)
        h = h + r
        h_all[pl.ds(i * R, R), :] = h

        @pl.when(i == 0)
        def _():
            acc_ref[...] = jnp.zeros_like(acc_ref)

        acc_ref[0:1, :] += jnp.sum(h, axis=0, keepdims=True)
        acc_ref[1:2, :] += jnp.sum(h * h, axis=0, keepdims=True)

    @pl.when(ph == 1)
    def _():
        mean = acc_ref[0:1, :] * (1.0 / N)
        var = acc_ref[1:2, :] * (1.0 / N) - mean * mean
        inv = lax.rsqrt(var + EPS)
        h = h_all[pl.ds(i * R, R), :]
        o_ref[...] = (h - mean) * (inv * g_ref[...]) + bt_ref[...]


def kernel(feats, edge_index, W, b, W_res, b_res, gamma, beta):
    src = edge_index[0].astype(jnp.int32)
    dst = edge_index[1].astype(jnp.int32)
    pad = EP - E
    src_p = jnp.concatenate([src, jnp.zeros((pad,), jnp.int32)])
    dst_p = jnp.concatenate([dst, jnp.full((pad,), N, jnp.int32)])
    zeros = jnp.zeros((INIT_ROWS, D), jnp.float32)

    feats_bf = feats.astype(jnp.bfloat16)
    feats_bf = feats_bf.reshape(N, D // 32, 2, 16).transpose(0, 1, 3, 2)
    feats_i32 = lax.bitcast_convert_type(
        feats_bf.reshape(N, D // 2, 2), jnp.int32)

    parts = _sc_segment_sum(src_p, dst_p, feats_i32, zeros)
    p0, p1 = parts[0], parts[1]

    blk = lambda ph, i: (i * (1 - ph), 0)
    out_blk = lambda ph, i: (i, 0)
    full = lambda ph, i: (0, 0)
    out = pl.pallas_call(
        _tc_fused_body,
        grid=(2, NBLK),
        in_specs=[
            pl.BlockSpec((R, D), blk),
            pl.BlockSpec((R, D), blk),
            pl.BlockSpec((R, D), blk),
            pl.BlockSpec((D, D), full),
            pl.BlockSpec((1, D), full),
            pl.BlockSpec((D, D), full),
            pl.BlockSpec((1, D), full),
            pl.BlockSpec((1, D), full),
            pl.BlockSpec((1, D), full),
        ],
        out_specs=pl.BlockSpec((R, D), out_blk),
        out_shape=jax.ShapeDtypeStruct((N, D), jnp.float32),
        scratch_shapes=[
            pltpu.VMEM((N, D), jnp.float32),
            pltpu.VMEM((2, D), jnp.float32),
        ],
    )(p0, p1, feats, W, b.reshape(1, D), W_res, b_res.reshape(1, D),
      gamma.reshape(1, D), beta.reshape(1, D))
    return out


# speedup vs baseline: 1.8863x; 1.0581x over previous
import functools

import numpy as np

import jax
import jax.numpy as jnp
from jax import lax
from jax.experimental import pallas as pl
from jax.experimental.pallas import tpu as pltpu
from jax.experimental.pallas import tpu_sc as plsc

N = 10000
E = 320000
D = 128
EPS = 1e-5

NC = 2
NS = 16
NW = NC * NS
C = 128

NCHUNK = E // C
CHUNKS_PER_W = NCHUNK // NW
EXTRA = NCHUNK - CHUNKS_PER_W * NW
EPW = CHUNKS_PER_W * C
NP = 10016
INIT_ROWS = NP // 4
OUT_ROWS = 632
OUT_TAIL = N - 15 * OUT_ROWS


def _sc_segment_sum(src_p, dst_p, feats_i32, zeros):
    mesh = plsc.VectorSubcoreMesh(core_axis_name="c", subcore_axis_name="s")
    D2 = D // 2

    @functools.partial(
        pl.kernel,
        out_type=jax.ShapeDtypeStruct((NC, N, D), jnp.float32),
        mesh=mesh,
        compiler_params=pltpu.CompilerParams(use_tc_tiling_on_sc=False),
        scratch_types=[
            pltpu.VMEM((C,), jnp.int32),
            pltpu.VMEM((C,), jnp.int32),
            pltpu.VMEM((C,), jnp.int32),
            pltpu.VMEM((C,), jnp.int32),
            pltpu.VMEM((C, D2), jnp.int32),
            pltpu.VMEM((C, D2), jnp.int32),
            pltpu.VMEM((C, D), jnp.float32),
            pltpu.VMEM((C, D), jnp.float32),
            pltpu.VMEM_SHARED((NP, D), jnp.float32),
            pltpu.SemaphoreType.DMA,
            pltpu.SemaphoreType.DMA,
            pltpu.SemaphoreType.DMA,
            pltpu.SemaphoreType.DMA,
        ],
    )
    def seg_sum(src_hbm, dst_hbm, feats_hbm, zeros_hbm, out_hbm,
                src0_v, src1_v, dst0_v, dst1_v, bf0, bf1, rf0, rf1,
                acc_sh, sem0, sem1, sem2, sem3):
        cid = lax.axis_index("c")
        sid = lax.axis_index("s")
        wid = sid * NC + cid
        @pl.when(sid < 4)
        def _():
            pltpu.sync_copy(zeros_hbm,
                            acc_sh.at[pl.ds(sid * INIT_ROWS, INIT_ROWS)])

        plsc.subcore_barrier()

        base = wid * EPW

        def unpack_rows(bf_ref, rf_ref):
            @pl.loop(0, C, unroll=4)
            def _(r):
                for g in range(D2 // 16):
                    v = bf_ref[r, pl.ds(16 * g, 16)]
                    a = lax.bitcast_convert_type(v << 16, jnp.float32)
                    b2 = lax.bitcast_convert_type(
                        v & jnp.int32(-65536), jnp.float32)
                    rf_ref[r, pl.ds(32 * g, 16)] = a
                    rf_ref[r, pl.ds(32 * g + 16, 16)] = b2

        pltpu.sync_copy(src_hbm.at[pl.ds(base, C)], src0_v)
        pltpu.async_copy(feats_hbm.at[src0_v], bf0, sem0)
        NPAIR = CHUNKS_PER_W // 2

        @pl.loop(0, NPAIR)
        def _(p):
            off0 = base + 2 * p * C
            pltpu.sync_copy(src_hbm.at[pl.ds(off0 + C, C)], src1_v)
            pltpu.async_copy(feats_hbm.at[src1_v], bf1, sem1)
            pltpu.make_async_copy(feats_hbm.at[src0_v], bf0, sem0).wait()

            @pl.when(p > 0)
            def _():
                pltpu.make_async_copy(rf0, acc_sh.at[dst0_v], sem2).wait()

            unpack_rows(bf0, rf0)
            pltpu.sync_copy(dst_hbm.at[pl.ds(off0, C)], dst0_v)
            pltpu.make_async_copy(rf0, acc_sh.at[dst0_v], sem2).start(
                add=True)

            @pl.when(p < NPAIR - 1)
            def _():
                pltpu.sync_copy(src_hbm.at[pl.ds(off0 + 2 * C, C)], src0_v)
                pltpu.async_copy(feats_hbm.at[src0_v], bf0, sem0)

            pltpu.make_async_copy(feats_hbm.at[src1_v], bf1, sem1).wait()

            @pl.when(p > 0)
            def _():
                pltpu.make_async_copy(rf1, acc_sh.at[dst1_v], sem3).wait()

            unpack_rows(bf1, rf1)
            pltpu.sync_copy(dst_hbm.at[pl.ds(off0 + C, C)], dst1_v)
            pltpu.make_async_copy(rf1, acc_sh.at[dst1_v], sem3).start(
                add=True)

        pltpu.make_async_copy(rf0, acc_sh.at[dst0_v], sem2).wait()
        pltpu.make_async_copy(rf1, acc_sh.at[dst1_v], sem3).wait()

        @pl.when(wid < EXTRA)
        def _():
            off = (CHUNKS_PER_W * NW + wid) * C
            pltpu.sync_copy(src_hbm.at[pl.ds(off, C)], src0_v)
            pltpu.async_copy(feats_hbm.at[src0_v], bf0, sem0).wait()
            unpack_rows(bf0, rf0)
            pltpu.sync_copy(dst_hbm.at[pl.ds(off, C)], dst0_v)
            pltpu.sync_copy(rf0, acc_sh.at[dst0_v], add=True)

        plsc.subcore_barrier()

        @pl.when(sid < NS - 1)
        def _():
            pltpu.sync_copy(acc_sh.at[pl.ds(sid * OUT_ROWS, OUT_ROWS)],
                            out_hbm.at[cid, pl.ds(sid * OUT_ROWS, OUT_ROWS)])

        @pl.when(sid == NS - 1)
        def _():
            pltpu.sync_copy(acc_sh.at[pl.ds((NS - 1) * OUT_ROWS, OUT_TAIL)],
                            out_hbm.at[cid, pl.ds((NS - 1) * OUT_ROWS, OUT_TAIL)])

    return seg_sum(src_p, dst_p, feats_i32, zeros)


R = 1000
NBLK = N // R


def _tc_fused_body(p0_ref, p1_ref, f_ref, w_ref, b_ref, wr_ref, br_ref,
                   g_ref, bt_ref, o_ref, h_all, acc_ref):
    ph = pl.program_id(0)
    i = pl.program_id(1)

    @pl.when(ph == 0)
    def _():
        agg = p0_ref[...] + p1_ref[...]
        h = jnp.dot(agg, w_ref[...], preferred_element_type=jnp.float32)
        h = jnp.maximum(h + b_ref[...], 0.0)
        r = jnp.dot(f_ref[...], wr_ref[...],
                    preferred_element_type=jnp.float32)
        r = jnp.maximum(r + br_ref[...], 0.0)
        h = h + r
        h_all[pl.ds(i * R, R), :] = h

        @pl.when(i == 0)
        def _():
            acc_ref[...] = jnp.zeros_like(acc_ref)

        acc_ref[0:1, :] += jnp.sum(h, axis=0, keepdims=True)
        acc_ref[1:2, :] += jnp.sum(h * h, axis=0, keepdims=True)

    @pl.when(ph == 1)
    def _():
        mean = acc_ref[0:1, :] * (1.0 / N)
        var = acc_ref[1:2, :] * (1.0 / N) - mean * mean
        inv = lax.rsqrt(var + EPS)
        h = h_all[pl.ds(i * R, R), :]
        o_ref[...] = (h - mean) * (inv * g_ref[...]) + bt_ref[...]


def kernel(feats, edge_index, W, b, W_res, b_res, gamma, beta):
    src_p = edge_index[0].astype(jnp.int32)
    dst_p = edge_index[1].astype(jnp.int32)
    zeros = jnp.zeros((INIT_ROWS, D), jnp.float32)

    feats_bf = feats.astype(jnp.bfloat16)
    feats_bf = feats_bf.reshape(N, D // 32, 2, 16).transpose(0, 1, 3, 2)
    feats_i32 = lax.bitcast_convert_type(
        feats_bf.reshape(N, D // 2, 2), jnp.int32)

    parts = _sc_segment_sum(src_p, dst_p, feats_i32, zeros)
    p0, p1 = parts[0], parts[1]

    blk = lambda ph, i: (i * (1 - ph), 0)
    out_blk = lambda ph, i: (i, 0)
    full = lambda ph, i: (0, 0)
    out = pl.pallas_call(
        _tc_fused_body,
        grid=(2, NBLK),
        in_specs=[
            pl.BlockSpec((R, D), blk),
            pl.BlockSpec((R, D), blk),
            pl.BlockSpec((R, D), blk),
            pl.BlockSpec((D, D), full),
            pl.BlockSpec((1, D), full),
            pl.BlockSpec((D, D), full),
            pl.BlockSpec((1, D), full),
            pl.BlockSpec((1, D), full),
            pl.BlockSpec((1, D), full),
        ],
        out_specs=pl.BlockSpec((R, D), out_blk),
        out_shape=jax.ShapeDtypeStruct((N, D), jnp.float32),
        scratch_shapes=[
            pltpu.VMEM((N, D), jnp.float32),
            pltpu.VMEM((2, D), jnp.float32),
        ],
    )(p0, p1, feats, W, b.reshape(1, D), W_res, b_res.reshape(1, D),
      gamma.reshape(1, D), beta.reshape(1, D))
    return out
